# Initial kernel scaffold; baseline (speedup 1.0000x reference)
import jax
import jax.numpy as jnp
from jax.experimental import pallas as pl


def kernel(x, W1, W2, W3, W4, W5, f1aw, f1ab, f1bw, f1bb, f1cw, f1cb, f2aw, f2ab, f2bw, f2bb, f2cw, f2cb):
    raise NotImplementedError("write your pallas kernel here")


# speedup vs baseline: 1.9713x; 1.9713x over previous
import functools

import jax
import jax.numpy as jnp
from jax.experimental import pallas as pl
from jax.experimental.pallas import tpu as pltpu

_N = 2048
_K = 20
_B = 8
_R = 256


def _edge_kernel(x_ref, wt_ref, m_ref, s1_ref, s2_ref):
    XT = x_ref[0]
    WT = wt_ref[...]
    xx = jnp.sum(XT * XT, axis=1, keepdims=True)
    xxr = jnp.reshape(xx, (1, _N))
    O = WT.shape[1]
    iota = jax.lax.broadcasted_iota(jnp.int32, (_R, _N), 1)
    neg_inf = jnp.float32(-jnp.inf)

    s1 = jnp.zeros((1, O), jnp.float32)
    s2 = jnp.zeros((1, O), jnp.float32)
    for r in range(_N // _R):
        Xb = XT[r * _R:(r + 1) * _R]
        xxb = xx[r * _R:(r + 1) * _R]
        inner = -2.0 * jax.lax.dot_general(
            Xb, XT, (((1,), (1,)), ((), ())),
            preferred_element_type=jnp.float32)
        pd = -xxb - inner - xxr

        def body(_, c):
            pdc, g1, g2, gm = c
            mx = jnp.max(pdc, axis=1, keepdims=True)
            cand = jnp.where(pdc == mx, iota, _N)
            j = jnp.min(cand, axis=1, keepdims=True)
            oh = iota == j
            Xj = jnp.dot(oh.astype(jnp.float32), XT,
                         preferred_element_type=jnp.float32,
                         precision=jax.lax.Precision.HIGHEST)
            E = jnp.concatenate([Xj - Xb, Xb], axis=1)
            g = jnp.dot(E, WT, preferred_element_type=jnp.float32)
            return (jnp.where(oh, neg_inf, pdc),
                    g1 + g, g2 + g * g, jnp.maximum(gm, g))

        z = jnp.zeros((_R, O), jnp.float32)
        _, g1, g2, gm = jax.lax.fori_loop(
            0, _K, body, (pd, z, z, jnp.full((_R, O), neg_inf, jnp.float32)))

        m_ref[0, r * _R:(r + 1) * _R, :] = gm
        s1 = s1 + jnp.sum(g1, axis=0, keepdims=True)
        s2 = s2 + jnp.sum(g2, axis=0, keepdims=True)
    s1_ref[0] = s1
    s2_ref[0] = s2


def _edge_layer(x, wt):
    C = x.shape[-1]
    O = wt.shape[-1]
    return pl.pallas_call(
        _edge_kernel,
        grid=(_B,),
        in_specs=[
            pl.BlockSpec((1, _N, C), lambda b: (b, 0, 0)),
            pl.BlockSpec((2 * C, O), lambda b: (0, 0)),
        ],
        out_specs=[
            pl.BlockSpec((1, _N, O), lambda b: (b, 0, 0)),
            pl.BlockSpec((1, 1, O), lambda b: (b, 0, 0)),
            pl.BlockSpec((1, 1, O), lambda b: (b, 0, 0)),
        ],
        out_shape=[
            jax.ShapeDtypeStruct((_B, _N, O), jnp.float32),
            jax.ShapeDtypeStruct((_B, 1, O), jnp.float32),
            jax.ShapeDtypeStruct((_B, 1, O), jnp.float32),
        ],
    )(x, wt)


def _bn_kernel(m_ref, s1_ref, s2_ref, o_ref, *, cnt):
    S1 = jnp.sum(s1_ref[...], axis=0)
    S2 = jnp.sum(s2_ref[...], axis=0)
    mu = S1 / cnt
    sd = jnp.sqrt(S2 / cnt - mu * mu + 1e-5)
    y = (m_ref[0] - mu) / sd
    o_ref[0] = jnp.where(y > 0, y, 0.2 * y)


def _bn_finalize(m, s1, s2, cnt):
    O = m.shape[-1]
    return pl.pallas_call(
        functools.partial(_bn_kernel, cnt=cnt),
        grid=(_B,),
        in_specs=[
            pl.BlockSpec((1, _N, O), lambda b: (b, 0, 0)),
            pl.BlockSpec((_B, 1, O), lambda b: (0, 0, 0)),
            pl.BlockSpec((_B, 1, O), lambda b: (0, 0, 0)),
        ],
        out_specs=pl.BlockSpec((1, _N, O), lambda b: (b, 0, 0)),
        out_shape=jax.ShapeDtypeStruct((_B, _N, O), jnp.float32),
    )(m, s1, s2)


def _k5_kernel(x1_ref, x2_ref, x3_ref, x4_ref, wa_ref, wb_ref, wc_ref, wd_ref,
               my_ref, s1_ref, s2_ref):
    dot = functools.partial(jnp.dot, preferred_element_type=jnp.float32)
    Y = (dot(x1_ref[0], wa_ref[...]) + dot(x2_ref[0], wb_ref[...])
         + dot(x3_ref[0], wc_ref[...]) + dot(x4_ref[0], wd_ref[...]))
    my_ref[0] = jnp.max(Y, axis=0, keepdims=True)
    s1_ref[0] = jnp.sum(Y, axis=0, keepdims=True)
    s2_ref[0] = jnp.sum(Y * Y, axis=0, keepdims=True)


def _k5(x1, x2, x3, x4, w5_parts):
    specs = [pl.BlockSpec((1, _N, x.shape[-1]), lambda b: (b, 0, 0))
             for x in (x1, x2, x3, x4)]
    specs += [pl.BlockSpec(w.shape, lambda b: (0, 0)) for w in w5_parts]
    return pl.pallas_call(
        _k5_kernel,
        grid=(_B,),
        in_specs=specs,
        out_specs=[pl.BlockSpec((1, 1, 512), lambda b: (b, 0, 0))] * 3,
        out_shape=[jax.ShapeDtypeStruct((_B, 1, 512), jnp.float32)] * 3,
    )(x1, x2, x3, x4, *w5_parts)


def _dec_kernel(my_ref, s1_ref, s2_ref, pts_ref,
                f1az_ref, f1ap_ref, f1ab_ref, f1bw_ref, f1bb_ref,
                f1cw_ref, f1cb_ref,
                f2az_ref, f2ap_ref, f2ab_ref, f2bw_ref, f2bb_ref,
                f2cw_ref, f2cb_ref,
                out_ref, feat_ref, *, cnt):
    S1 = jnp.sum(s1_ref[...], axis=0)
    S2 = jnp.sum(s2_ref[...], axis=0)
    mu = S1 / cnt
    sd = jnp.sqrt(S2 / cnt - mu * mu + 1e-5)
    y = (my_ref[0] - mu) / sd
    feat = jnp.where(y > 0, y, 0.2 * y)
    feat_ref[0] = feat

    dot = functools.partial(jnp.dot, preferred_element_type=jnp.float32)
    v1 = dot(feat, f1az_ref[...]) + f1ab_ref[...]
    h = jax.nn.relu(v1 + dot(pts_ref[...], f1ap_ref[...]))
    h = jax.nn.relu(dot(h, f1bw_ref[...]) + f1bb_ref[...])
    fold1 = dot(h, f1cw_ref[...]) + f1cb_ref[...]
    v2 = dot(feat, f2az_ref[...]) + f2ab_ref[...]
    h = jax.nn.relu(v2 + dot(fold1, f2ap_ref[...]))
    h = jax.nn.relu(dot(h, f2bw_ref[...]) + f2bb_ref[...])
    out_ref[0] = dot(h, f2cw_ref[...]) + f2cb_ref[...]


def _decoder(my, s1, s2, pts, wts):
    specs = [pl.BlockSpec((1, 1, 512), lambda b: (b, 0, 0)),
             pl.BlockSpec((_B, 1, 512), lambda b: (0, 0, 0)),
             pl.BlockSpec((_B, 1, 512), lambda b: (0, 0, 0)),
             pl.BlockSpec(pts.shape, lambda b: (0, 0))]
    specs += [pl.BlockSpec(w.shape, lambda b: (0, 0)) for w in wts]
    return pl.pallas_call(
        functools.partial(_dec_kernel, cnt=float(_B * _N)),
        grid=(_B,),
        in_specs=specs,
        out_specs=[
            pl.BlockSpec((1, _N, 128), lambda b: (b, 0, 0)),
            pl.BlockSpec((1, 1, 512), lambda b: (b, 0, 0)),
        ],
        out_shape=[
            jax.ShapeDtypeStruct((_B, _N, 128), jnp.float32),
            jax.ShapeDtypeStruct((_B, 1, 512), jnp.float32),
        ],
    )(my, s1, s2, pts, *wts)


def _build_grid_pts():
    import numpy as np
    xs = np.linspace(-0.3, 0.3, 45)
    grid = np.stack(np.meshgrid(xs, xs, indexing="ij"), axis=-1)
    pts = grid.reshape(-1, 2).astype(np.float32)
    out = np.zeros((_N, 128), np.float32)
    out[:2025, :2] = pts
    return jnp.asarray(out)


def _edge_wt(W):
    return jnp.transpose(W)


def kernel(x, W1, W2, W3, W4, W5, f1aw, f1ab, f1bw, f1bb, f1cw, f1cb,
           f2aw, f2ab, f2bw, f2bb, f2cw, f2cb):
    cnt_e = float(_B * _N * _K)

    xp = jnp.pad(x, ((0, 0), (0, 0), (0, 5)))
    w1t = jnp.zeros((16, 64), jnp.float32)
    w1t = w1t.at[0:3].set(jnp.transpose(W1[:, :3]))
    w1t = w1t.at[8:11].set(jnp.transpose(W1[:, 3:]))
    m1, a1, b1 = _edge_layer(xp, w1t)
    x1 = _bn_finalize(m1, a1, b1, cnt_e)

    m2, a2, b2 = _edge_layer(x1, _edge_wt(W2))
    x2 = _bn_finalize(m2, a2, b2, cnt_e)

    m3, a3, b3 = _edge_layer(x2, _edge_wt(W3))
    x3 = _bn_finalize(m3, a3, b3, cnt_e)

    m4, a4, b4 = _edge_layer(x3, _edge_wt(W4))
    x4 = _bn_finalize(m4, a4, b4, cnt_e)

    w5t = jnp.transpose(W5)
    w5_parts = (w5t[:64], w5t[64:128], w5t[128:256], w5t[256:512])
    my, s1, s2 = _k5(x1, x2, x3, x4, w5_parts)

    pts = _build_grid_pts()
    f1az = jnp.transpose(f1aw[:, :512])
    f1ap = jnp.pad(jnp.transpose(f1aw[:, 512:]), ((0, 126), (0, 0)))
    f1cwt = jnp.pad(jnp.transpose(f1cw), ((0, 0), (0, 125)))
    f1cbp = jnp.pad(f1cb, (0, 125)).reshape(1, 128)
    f2az = jnp.transpose(f2aw[:, :512])
    f2ap = jnp.pad(jnp.transpose(f2aw[:, 512:]), ((0, 125), (0, 0)))
    f2cwt = jnp.pad(jnp.transpose(f2cw), ((0, 0), (0, 125)))
    f2cbp = jnp.pad(f2cb, (0, 125)).reshape(1, 128)
    wts = (f1az, f1ap, f1ab.reshape(1, 512), jnp.transpose(f1bw),
           f1bb.reshape(1, 512), f1cwt, f1cbp,
           f2az, f2ap, f2ab.reshape(1, 512), jnp.transpose(f2bw),
           f2bb.reshape(1, 512), f2cwt, f2cbp)
    out, feat = _decoder(my, s1, s2, pts, wts)
    return out[:, :2025, :3], feat
